# baseline (speedup 1.0000x reference)
import functools

import jax
import jax.numpy as jnp
from jax import lax
from jax.experimental import pallas as pl
from jax.experimental.pallas import tpu as pltpu
from jax.experimental.pallas import tpu_sc as plsc

N = 819200
M = 32
L = 512
LANES = 16

_info = plsc.get_sparse_core_info()
NC = _info.num_cores
NS = _info.num_subcores
NW = NC * NS

ROWS_PER_W = N // NW
CH = 1024
NCH = ROWS_PER_W // CH

PACK = 128 // M
NP = N // PACK
COMB = 2048
TCBLK = PACK * COMB
NBLK = N // TCBLK


def _sc_gather(embedding, tok):
  mesh = plsc.VectorSubcoreMesh(core_axis_name="c", subcore_axis_name="s")

  @functools.partial(
      pl.kernel,
      mesh=mesh,
      out_type=jax.ShapeDtypeStruct((N, M), jnp.float32),
      scratch_types=[
          pltpu.VMEM((CH,), jnp.int32),
          pltpu.VMEM((CH,), jnp.int32),
          pltpu.VMEM((CH, M), jnp.float32),
          pltpu.VMEM((CH, M), jnp.float32),
          pltpu.VMEM((CH,), jnp.int32),
          pltpu.VMEM((CH,), jnp.int32),
          pltpu.VMEM((CH,), jnp.int32),
          pltpu.SemaphoreType.DMA,
          pltpu.SemaphoreType.DMA,
          pltpu.SemaphoreType.DMA,
          pltpu.SemaphoreType.DMA,
      ],
      compiler_params=pltpu.CompilerParams(use_tc_tiling_on_sc=False),
  )
  def k(emb_hbm, tok_hbm, out_hbm, idx0, idx1, rows0, rows1, di0, di1, patt,
        sg0, sg1, so0, so1):
    wid = lax.axis_index("s") * NC + lax.axis_index("c")
    slab = wid * ROWS_PER_W
    idx = (idx0, idx1)
    rows = (rows0, rows1)
    di = (di0, di1)
    sg = (sg0, sg1)
    so = (so0, so1)

    lane = lax.iota(jnp.int32, LANES)

    def patt_body(r, _):
      patt[pl.ds(r * LANES, LANES)] = (r * LANES + lane) * PACK
      return _

    lax.fori_loop(0, CH // LANES, patt_body, None)

    def fill_di(dref, d0):
      def body(r, _):
        dref[pl.ds(r * LANES, LANES)] = patt[pl.ds(r * LANES, LANES)] + d0
        return _

      lax.fori_loop(0, CH // LANES, body, None)

    def dst_base(c):
      s = slab + c * CH
      return ((s & ~(TCBLK - 1)) + PACK * (s & (COMB - 1))
              + ((s >> 11) & (PACK - 1)))

    gath = {}
    wout = {}
    pltpu.sync_copy(tok_hbm.at[pl.ds(slab, CH)], idx0)
    gath[0] = pltpu.async_copy(emb_hbm.at[idx0], rows0, sg0)
    for c in range(NCH):
      b = c & 1
      if c + 1 < NCH:
        nb = 1 - b
        pltpu.sync_copy(tok_hbm.at[pl.ds(slab + (c + 1) * CH, CH)], idx[nb])
        if c >= 1:
          wout[c - 1].wait()
        gath[c + 1] = pltpu.async_copy(emb_hbm.at[idx[nb]], rows[nb], sg[nb])
      fill_di(di[b], dst_base(c))
      gath[c].wait()
      wout[c] = pltpu.async_copy(rows[b], out_hbm.at[di[b]], so[b])
    wout[NCH - 2].wait()
    wout[NCH - 1].wait()

  return k(embedding, tok)


def _tc_finish(x2, bp4, apat, kmat):
  def body(x_ref, bp_ref, ap_ref, k_ref, o_ref):
    bpt = jnp.tile(bp_ref[...], (COMB // L, 1))
    t = x_ref[...] * bpt
    th = t.astype(jnp.bfloat16).astype(jnp.float32)
    tl = t - th
    dn = (((1,), (1,)), ((), ()))
    ut = (lax.dot_general(k_ref[...], th, dimension_numbers=dn,
                          preferred_element_type=jnp.float32)
          + lax.dot_general(k_ref[...], tl, dimension_numbers=dn,
                            preferred_element_type=jnp.float32))
    ap = ap_ref[...]
    for q in range(PACK):
      o_ref[:, pl.ds(q * COMB, COMB)] = ut[q * M:(q + 1) * M, :] * ap

  return pl.pallas_call(
      body,
      grid=(NBLK,),
      in_specs=[
          pl.BlockSpec((COMB, 128), lambda i: (i, 0)),
          pl.BlockSpec((L, 128), lambda i: (0, 0)),
          pl.BlockSpec((M, COMB), lambda i: (0, 0)),
          pl.BlockSpec((128, 128), lambda i: (0, 0)),
      ],
      out_specs=pl.BlockSpec((M, TCBLK), lambda i: (0, i)),
      out_shape=jax.ShapeDtypeStruct((M, N), jnp.float32),
  )(x2, bp4, apat, kmat)


def kernel(k_tensor, token_indices, embedding, Acoeff, Bbasis):
  del k_tensor
  tok = token_indices.astype(jnp.int32)
  xg = _sc_gather(embedding, tok)
  x2 = xg.reshape(NP, 128)
  bp4 = jnp.tile(Bbasis, (1, PACK))
  apat = jnp.tile(Acoeff, (1, COMB // L))
  seg = jnp.arange(128, dtype=jnp.int32) // M
  kmat = (seg[:, None] == seg[None, :]).astype(jnp.float32)
  return _tc_finish(x2, bp4, apat, kmat).T


# speedup vs baseline: 1.1188x; 1.1188x over previous
import functools

import jax
import jax.numpy as jnp
from jax import lax
from jax.experimental import pallas as pl
from jax.experimental.pallas import tpu as pltpu
from jax.experimental.pallas import tpu_sc as plsc

N = 819200
M = 32
L = 512
LANES = 16

_info = plsc.get_sparse_core_info()
NC = _info.num_cores
NS = _info.num_subcores
NW = NC * NS

ROWS_PER_W = N // NW
CH = 1024
NCH = ROWS_PER_W // CH

PACK = 128 // M
NP = N // PACK
COMB = 4096
TCBLK = PACK * COMB
NBLK = N // TCBLK


def _sc_gather(embedding, tok):
  mesh = plsc.VectorSubcoreMesh(core_axis_name="c", subcore_axis_name="s")

  @functools.partial(
      pl.kernel,
      mesh=mesh,
      out_type=jax.ShapeDtypeStruct((N, M), jnp.float32),
      scratch_types=[
          pltpu.VMEM((CH,), jnp.int32),
          pltpu.VMEM((CH,), jnp.int32),
          pltpu.VMEM((CH, M), jnp.float32),
          pltpu.VMEM((CH, M), jnp.float32),
          pltpu.VMEM((CH,), jnp.int32),
          pltpu.VMEM((CH,), jnp.int32),
          pltpu.VMEM((CH,), jnp.int32),
          pltpu.SemaphoreType.DMA,
          pltpu.SemaphoreType.DMA,
          pltpu.SemaphoreType.DMA,
          pltpu.SemaphoreType.DMA,
      ],
      compiler_params=pltpu.CompilerParams(use_tc_tiling_on_sc=False),
  )
  def k(emb_hbm, tok_hbm, out_hbm, idx0, idx1, rows0, rows1, di0, di1, patt,
        sg0, sg1, so0, so1):
    wid = lax.axis_index("s") * NC + lax.axis_index("c")
    slab = wid * ROWS_PER_W
    idx = (idx0, idx1)
    rows = (rows0, rows1)
    di = (di0, di1)
    sg = (sg0, sg1)
    so = (so0, so1)

    lane = lax.iota(jnp.int32, LANES)

    def patt_body(r, _):
      patt[pl.ds(r * LANES, LANES)] = (r * LANES + lane) * PACK
      return _

    lax.fori_loop(0, CH // LANES, patt_body, None)

    def fill_di(dref, d0):
      def body(r, _):
        dref[pl.ds(r * LANES, LANES)] = patt[pl.ds(r * LANES, LANES)] + d0
        return _

      lax.fori_loop(0, CH // LANES, body, None)

    def dst_base(c):
      s = slab + c * CH
      comb_shift = COMB.bit_length() - 1
      return ((s & ~(TCBLK - 1)) + PACK * (s & (COMB - 1))
              + ((s >> comb_shift) & (PACK - 1)))

    gath = {}
    wout = {}
    pltpu.sync_copy(tok_hbm.at[pl.ds(slab, CH)], idx0)
    gath[0] = pltpu.async_copy(emb_hbm.at[idx0], rows0, sg0)
    for c in range(NCH):
      b = c & 1
      if c + 1 < NCH:
        nb = 1 - b
        pltpu.sync_copy(tok_hbm.at[pl.ds(slab + (c + 1) * CH, CH)], idx[nb])
        if c >= 1:
          wout[c - 1].wait()
        gath[c + 1] = pltpu.async_copy(emb_hbm.at[idx[nb]], rows[nb], sg[nb])
      fill_di(di[b], dst_base(c))
      gath[c].wait()
      wout[c] = pltpu.async_copy(rows[b], out_hbm.at[di[b]], so[b])
    wout[NCH - 2].wait()
    wout[NCH - 1].wait()

  return k(embedding, tok)


def _tc_finish(x2, bp4, apat, kmat):
  def body(x_ref, bp_ref, ap_ref, k_ref, o_ref):
    bpt = jnp.tile(bp_ref[...], (COMB // L, 1))
    t = x_ref[...] * bpt
    th = t.astype(jnp.bfloat16).astype(jnp.float32)
    tl = t - th
    dn = (((1,), (1,)), ((), ()))
    ut = (lax.dot_general(k_ref[...], th, dimension_numbers=dn,
                          preferred_element_type=jnp.float32)
          + lax.dot_general(k_ref[...], tl, dimension_numbers=dn,
                            preferred_element_type=jnp.float32))
    ap = ap_ref[...]
    for q in range(PACK):
      o_ref[:, pl.ds(q * COMB, COMB)] = ut[q * M:(q + 1) * M, :] * ap

  return pl.pallas_call(
      body,
      grid=(NBLK,),
      in_specs=[
          pl.BlockSpec((COMB, 128), lambda i: (i, 0)),
          pl.BlockSpec((L, 128), lambda i: (0, 0)),
          pl.BlockSpec((M, COMB), lambda i: (0, 0)),
          pl.BlockSpec((128, 128), lambda i: (0, 0)),
      ],
      out_specs=pl.BlockSpec((M, TCBLK), lambda i: (0, i)),
      out_shape=jax.ShapeDtypeStruct((M, N), jnp.float32),
  )(x2, bp4, apat, kmat)


def kernel(k_tensor, token_indices, embedding, Acoeff, Bbasis):
  del k_tensor
  tok = token_indices.astype(jnp.int32)
  xg = _sc_gather(embedding, tok)
  x2 = xg.reshape(NP, 128)
  bp4 = jnp.tile(Bbasis, (1, PACK))
  apat = jnp.tile(Acoeff, (1, COMB // L))
  seg = jnp.arange(128, dtype=jnp.int32) // M
  kmat = (seg[:, None] == seg[None, :]).astype(jnp.float32)
  return _tc_finish(x2, bp4, apat, kmat).T


# speedup vs baseline: 1.1770x; 1.0521x over previous
import functools

import jax
import jax.numpy as jnp
from jax import lax
from jax.experimental import pallas as pl
from jax.experimental.pallas import tpu as pltpu
from jax.experimental.pallas import tpu_sc as plsc

N = 819200
M = 32
L = 512
LANES = 16

_info = plsc.get_sparse_core_info()
NC = _info.num_cores
NS = _info.num_subcores
NW = NC * NS

ROWS_PER_W = N // NW
CH = 1024
NCH = ROWS_PER_W // CH

PACK = 128 // M
NP = N // PACK
COMB = 8192
TCBLK = PACK * COMB
NBLK = N // TCBLK


def _sc_gather(embedding, tok):
  mesh = plsc.VectorSubcoreMesh(core_axis_name="c", subcore_axis_name="s")

  @functools.partial(
      pl.kernel,
      mesh=mesh,
      out_type=jax.ShapeDtypeStruct((N, M), jnp.float32),
      scratch_types=[
          pltpu.VMEM((CH,), jnp.int32),
          pltpu.VMEM((CH,), jnp.int32),
          pltpu.VMEM((CH, M), jnp.float32),
          pltpu.VMEM((CH, M), jnp.float32),
          pltpu.VMEM((CH,), jnp.int32),
          pltpu.VMEM((CH,), jnp.int32),
          pltpu.VMEM((CH,), jnp.int32),
          pltpu.SemaphoreType.DMA,
          pltpu.SemaphoreType.DMA,
          pltpu.SemaphoreType.DMA,
          pltpu.SemaphoreType.DMA,
      ],
      compiler_params=pltpu.CompilerParams(use_tc_tiling_on_sc=False),
  )
  def k(emb_hbm, tok_hbm, out_hbm, idx0, idx1, rows0, rows1, di0, di1, patt,
        sg0, sg1, so0, so1):
    wid = lax.axis_index("s") * NC + lax.axis_index("c")
    slab = wid * ROWS_PER_W
    idx = (idx0, idx1)
    rows = (rows0, rows1)
    di = (di0, di1)
    sg = (sg0, sg1)
    so = (so0, so1)

    lane = lax.iota(jnp.int32, LANES)

    def patt_body(r, _):
      patt[pl.ds(r * LANES, LANES)] = (r * LANES + lane) * PACK
      return _

    lax.fori_loop(0, CH // LANES, patt_body, None)

    def fill_di(dref, d0):
      def body(r, _):
        dref[pl.ds(r * LANES, LANES)] = patt[pl.ds(r * LANES, LANES)] + d0
        return _

      lax.fori_loop(0, CH // LANES, body, None)

    def dst_base(c):
      s = slab + c * CH
      comb_shift = COMB.bit_length() - 1
      return ((s & ~(TCBLK - 1)) + PACK * (s & (COMB - 1))
              + ((s >> comb_shift) & (PACK - 1)))

    gath = {}
    wout = {}
    pltpu.sync_copy(tok_hbm.at[pl.ds(slab, CH)], idx0)
    gath[0] = pltpu.async_copy(emb_hbm.at[idx0], rows0, sg0)
    for c in range(NCH):
      b = c & 1
      if c + 1 < NCH:
        nb = 1 - b
        pltpu.sync_copy(tok_hbm.at[pl.ds(slab + (c + 1) * CH, CH)], idx[nb])
        if c >= 1:
          wout[c - 1].wait()
        gath[c + 1] = pltpu.async_copy(emb_hbm.at[idx[nb]], rows[nb], sg[nb])
      fill_di(di[b], dst_base(c))
      gath[c].wait()
      wout[c] = pltpu.async_copy(rows[b], out_hbm.at[di[b]], so[b])
    wout[NCH - 2].wait()
    wout[NCH - 1].wait()

  return k(embedding, tok)


def _tc_finish(x2, bp4, apat, kmat):
  def body(x_ref, bp_ref, ap_ref, k_ref, o_ref):
    bpt = jnp.tile(bp_ref[...], (COMB // L, 1))
    t = x_ref[...] * bpt
    th = t.astype(jnp.bfloat16).astype(jnp.float32)
    tl = t - th
    dn = (((1,), (1,)), ((), ()))
    ut = (lax.dot_general(k_ref[...], th, dimension_numbers=dn,
                          preferred_element_type=jnp.float32)
          + lax.dot_general(k_ref[...], tl, dimension_numbers=dn,
                            preferred_element_type=jnp.float32))
    ap = ap_ref[...]
    for q in range(PACK):
      o_ref[:, pl.ds(q * COMB, COMB)] = ut[q * M:(q + 1) * M, :] * ap

  return pl.pallas_call(
      body,
      grid=(NBLK,),
      in_specs=[
          pl.BlockSpec((COMB, 128), lambda i: (i, 0)),
          pl.BlockSpec((L, 128), lambda i: (0, 0)),
          pl.BlockSpec((M, COMB), lambda i: (0, 0)),
          pl.BlockSpec((128, 128), lambda i: (0, 0)),
      ],
      out_specs=pl.BlockSpec((M, TCBLK), lambda i: (0, i)),
      out_shape=jax.ShapeDtypeStruct((M, N), jnp.float32),
  )(x2, bp4, apat, kmat)


def kernel(k_tensor, token_indices, embedding, Acoeff, Bbasis):
  del k_tensor
  tok = token_indices.astype(jnp.int32)
  xg = _sc_gather(embedding, tok)
  x2 = xg.reshape(NP, 128)
  bp4 = jnp.tile(Bbasis, (1, PACK))
  apat = jnp.tile(Acoeff, (1, COMB // L))
  seg = jnp.arange(128, dtype=jnp.int32) // M
  kmat = (seg[:, None] == seg[None, :]).astype(jnp.float32)
  return _tc_finish(x2, bp4, apat, kmat).T
